# Initial kernel scaffold; baseline (speedup 1.0000x reference)
import jax
import jax.numpy as jnp
from jax.experimental import pallas as pl


def kernel(timestamps, table):
    raise NotImplementedError("write your pallas kernel here")


# speedup vs baseline: 9.2214x; 9.2214x over previous
import functools

import jax
import jax.numpy as jnp
from jax import lax
from jax.experimental import pallas as pl
from jax.experimental.pallas import tpu as pltpu
from jax.experimental.pallas import tpu_sc as plsc

EMB = 128
IW = 128
K = 8
HK = 4
HROWS = HK * IW


def _gather_body(ts_hbm, table_hbm, out_hbm, idx_v, rows_v, sem, *, nc, steps_per_w):
    wid = lax.axis_index("s") * nc + lax.axis_index("c")
    row0 = wid * steps_per_w * K

    def step(t, carry):
        r = row0 + t * K
        pltpu.sync_copy(ts_hbm.at[pl.ds(r, K)], idx_v)
        for h in range(K // HK):
            descs = []
            for j in range(HK):
                descs.append(
                    pltpu.async_copy(
                        table_hbm.at[idx_v.at[h * HK + j]],
                        rows_v.at[pl.ds(j * IW, IW)],
                        sem,
                    )
                )
            for d in descs:
                d.wait()
            pltpu.sync_copy(rows_v, out_hbm.at[pl.ds((r + h * HK) * IW, HROWS)])
        return carry

    lax.fori_loop(0, steps_per_w, step, 0)


def kernel(timestamps, table):
    n, s = timestamps.shape
    b = n * s
    assert b % IW == 0
    idx2d = timestamps.reshape(b // IW, IW).astype(jnp.int32)

    info = plsc.get_sparse_core_info()
    nc, ns = info.num_cores, info.num_subcores
    nw = nc * ns
    assert (b // IW) % (nw * K) == 0
    steps_per_w = (b // IW) // (nw * K)

    mesh = plsc.VectorSubcoreMesh(core_axis_name="c", subcore_axis_name="s")
    out = pl.kernel(
        functools.partial(_gather_body, nc=nc, steps_per_w=steps_per_w),
        out_type=jax.ShapeDtypeStruct((b, EMB), jnp.float32),
        mesh=mesh,
        scratch_types=[
            pltpu.VMEM((K, IW), jnp.int32),
            pltpu.VMEM((HROWS, EMB), jnp.float32),
            pltpu.SemaphoreType.DMA,
        ],
    )(idx2d, table)
    return out.reshape(n, s, EMB)


# speedup vs baseline: 9.6217x; 1.0434x over previous
import functools

import jax
import jax.numpy as jnp
from jax import lax
from jax.experimental import pallas as pl
from jax.experimental.pallas import tpu as pltpu
from jax.experimental.pallas import tpu_sc as plsc

EMB = 128
IW = 128
K = 8
Q = 2
QROWS = Q * IW
NQ = K // Q


def _gather_body(ts_hbm, table_hbm, out_hbm, idx_v, buf_a, buf_b, sem_g,
                 sem_sa, sem_sb, *, nc, steps_per_w):
    wid = lax.axis_index("s") * nc + lax.axis_index("c")
    row0 = wid * steps_per_w * K
    bufs = (buf_a, buf_b)
    ssems = (sem_sa, sem_sb)

    def step(t, carry):
        r = row0 + t * K
        pltpu.sync_copy(ts_hbm.at[pl.ds(r, K)], idx_v)
        for h in range(NQ):
            buf = bufs[h % 2]
            ssem = ssems[h % 2]
            drain = lambda b=buf, s=ssem: pltpu.make_async_copy(
                b, out_hbm.at[pl.ds(0, QROWS)], s).wait()
            if h < 2:
                pl.when(t > 0)(drain)
            else:
                drain()
            descs = []
            for j in range(Q):
                descs.append(
                    pltpu.async_copy(
                        table_hbm.at[idx_v.at[h * Q + j]],
                        buf.at[pl.ds(j * IW, IW)],
                        sem_g,
                    )
                )
            for d in descs:
                d.wait()
            pltpu.async_copy(buf, out_hbm.at[pl.ds((r + h * Q) * IW, QROWS)], ssem)
        return carry

    lax.fori_loop(0, steps_per_w, step, 0)
    pltpu.make_async_copy(buf_a, out_hbm.at[pl.ds(0, QROWS)], sem_sa).wait()
    pltpu.make_async_copy(buf_b, out_hbm.at[pl.ds(0, QROWS)], sem_sb).wait()


def kernel(timestamps, table):
    n, s = timestamps.shape
    b = n * s
    assert b % IW == 0
    idx2d = timestamps.reshape(b // IW, IW).astype(jnp.int32)

    info = plsc.get_sparse_core_info()
    nc, ns = info.num_cores, info.num_subcores
    nw = nc * ns
    assert (b // IW) % (nw * K) == 0
    steps_per_w = (b // IW) // (nw * K)

    mesh = plsc.VectorSubcoreMesh(core_axis_name="c", subcore_axis_name="s")
    out = pl.kernel(
        functools.partial(_gather_body, nc=nc, steps_per_w=steps_per_w),
        out_type=jax.ShapeDtypeStruct((b, EMB), jnp.float32),
        mesh=mesh,
        scratch_types=[
            pltpu.VMEM((K, IW), jnp.int32),
            pltpu.VMEM((QROWS, EMB), jnp.float32),
            pltpu.VMEM((QROWS, EMB), jnp.float32),
            pltpu.SemaphoreType.DMA,
            pltpu.SemaphoreType.DMA,
            pltpu.SemaphoreType.DMA,
        ],
    )(idx2d, table)
    return out.reshape(n, s, EMB)


# speedup vs baseline: 9.7584x; 1.0142x over previous
import functools

import jax
import jax.numpy as jnp
from jax import lax
from jax.experimental import pallas as pl
from jax.experimental.pallas import tpu as pltpu
from jax.experimental.pallas import tpu_sc as plsc

EMB = 128
IW = 128
K = 8
NB = 4


def _gather_body(ts_hbm, table_hbm, out_hbm, idx_v, b0, b1, b2, b3,
                 g0, g1, g2, g3, s0, s1, s2, s3, *, nc, steps_per_w):
    wid = lax.axis_index("s") * nc + lax.axis_index("c")
    row0 = wid * steps_per_w * K
    bufs = (b0, b1, b2, b3)
    gsems = (g0, g1, g2, g3)
    ssems = (s0, s1, s2, s3)

    def drain_store(j):
        pltpu.make_async_copy(bufs[j], out_hbm.at[pl.ds(0, IW)], ssems[j]).wait()

    def fire_gather(j, idx_row):
        pltpu.async_copy(table_hbm.at[idx_v.at[idx_row]], bufs[j], gsems[j])

    def step(t, carry):
        r = row0 + t * K
        pltpu.sync_copy(ts_hbm.at[pl.ds(r, K)], idx_v)
        for j in range(NB):
            pl.when(t > 0)(lambda j=j: drain_store(j))
            fire_gather(j, j)
        for j in range(K):
            jb = j % NB
            pltpu.make_async_copy(
                table_hbm.at[idx_v.at[j]], bufs[jb], gsems[jb]).wait()
            pltpu.async_copy(bufs[jb], out_hbm.at[pl.ds((r + j) * IW, IW)],
                             ssems[jb])
            if j + NB < K:
                drain_store(jb)
                fire_gather(jb, j + NB)
        return carry

    lax.fori_loop(0, steps_per_w, step, 0)
    for j in range(NB):
        drain_store(j)


def kernel(timestamps, table):
    n, s = timestamps.shape
    b = n * s
    assert b % IW == 0
    idx2d = timestamps.reshape(b // IW, IW).astype(jnp.int32)

    info = plsc.get_sparse_core_info()
    nc, ns = info.num_cores, info.num_subcores
    nw = nc * ns
    assert (b // IW) % (nw * K) == 0
    steps_per_w = (b // IW) // (nw * K)

    mesh = plsc.VectorSubcoreMesh(core_axis_name="c", subcore_axis_name="s")
    out = pl.kernel(
        functools.partial(_gather_body, nc=nc, steps_per_w=steps_per_w),
        out_type=jax.ShapeDtypeStruct((b, EMB), jnp.float32),
        mesh=mesh,
        scratch_types=[pltpu.VMEM((K, IW), jnp.int32)]
        + [pltpu.VMEM((IW, EMB), jnp.float32) for _ in range(NB)]
        + [pltpu.SemaphoreType.DMA for _ in range(2 * NB)],
    )(idx2d, table)
    return out.reshape(n, s, EMB)


# speedup vs baseline: 17.8256x; 1.8267x over previous
import functools

import jax
import jax.numpy as jnp
from jax import lax
from jax.experimental import pallas as pl
from jax.experimental.pallas import tpu as pltpu
from jax.experimental.pallas import tpu_sc as plsc

EMB = 128
IW = 128
K = 8
NB = 4


def _gather_body(ts_hbm, table_hbm, out_hbm, table_sp, idx_v, b0, b1, b2, b3,
                 g0, g1, g2, g3, s0, s1, s2, s3, *, nc, steps_per_w):
    sid = lax.axis_index("s")
    wid = sid * nc + lax.axis_index("c")
    row0 = wid * steps_per_w * K
    bufs = (b0, b1, b2, b3)
    gsems = (g0, g1, g2, g3)
    ssems = (s0, s1, s2, s3)

    pl.when(sid == 0)(lambda: pltpu.sync_copy(table_hbm, table_sp))
    plsc.subcore_barrier()

    def drain_store(j):
        pltpu.make_async_copy(bufs[j], out_hbm.at[pl.ds(0, IW)], ssems[j]).wait()

    def fire_gather(j, idx_row):
        pltpu.async_copy(table_sp.at[idx_v.at[idx_row]], bufs[j], gsems[j])

    def step(t, carry):
        r = row0 + t * K
        pltpu.sync_copy(ts_hbm.at[pl.ds(r, K)], idx_v)
        for j in range(NB):
            pl.when(t > 0)(lambda j=j: drain_store(j))
            fire_gather(j, j)
        for j in range(K):
            jb = j % NB
            pltpu.make_async_copy(
                table_sp.at[idx_v.at[j]], bufs[jb], gsems[jb]).wait()
            pltpu.async_copy(bufs[jb], out_hbm.at[pl.ds((r + j) * IW, IW)],
                             ssems[jb])
            if j + NB < K:
                drain_store(jb)
                fire_gather(jb, j + NB)
        return carry

    lax.fori_loop(0, steps_per_w, step, 0)
    for j in range(NB):
        drain_store(j)


def kernel(timestamps, table):
    n, s = timestamps.shape
    b = n * s
    assert b % IW == 0
    idx2d = timestamps.reshape(b // IW, IW).astype(jnp.int32)

    info = plsc.get_sparse_core_info()
    nc, ns = info.num_cores, info.num_subcores
    nw = nc * ns
    assert (b // IW) % (nw * K) == 0
    steps_per_w = (b // IW) // (nw * K)

    mesh = plsc.VectorSubcoreMesh(core_axis_name="c", subcore_axis_name="s")
    out = pl.kernel(
        functools.partial(_gather_body, nc=nc, steps_per_w=steps_per_w),
        out_type=jax.ShapeDtypeStruct((b, EMB), jnp.float32),
        mesh=mesh,
        scratch_types=[pltpu.VMEM_SHARED(table.shape, jnp.float32),
                       pltpu.VMEM((K, IW), jnp.int32)]
        + [pltpu.VMEM((IW, EMB), jnp.float32) for _ in range(NB)]
        + [pltpu.SemaphoreType.DMA for _ in range(2 * NB)],
    )(idx2d, table)
    return out.reshape(n, s, EMB)


# speedup vs baseline: 19.0695x; 1.0698x over previous
import functools

import jax
import jax.numpy as jnp
from jax import lax
from jax.experimental import pallas as pl
from jax.experimental.pallas import tpu as pltpu
from jax.experimental.pallas import tpu_sc as plsc

EMB = 128
IW = 128
QK = 16
NB = 4
G = 2


def _gather_body(ts_hbm, table_hbm, out_hbm, table_sp, idx_v, b0, b1, b2, b3,
                 g0, g1, g2, g3, s0, s1, s2, s3, *, nc, steps_per_w):
    sid = lax.axis_index("s")
    wid = sid * nc + lax.axis_index("c")
    row0 = wid * steps_per_w * QK
    bufs = (b0, b1, b2, b3)
    gsems = (g0, g1, g2, g3)
    ssems = (s0, s1, s2, s3)

    pl.when(sid == 0)(lambda: pltpu.sync_copy(table_hbm, table_sp))
    plsc.subcore_barrier()

    def drain_store(b):
        pltpu.make_async_copy(bufs[b], out_hbm.at[pl.ds(0, IW)], ssems[b]).wait()

    def fire_gather(b, idx_row):
        pltpu.async_copy(table_sp.at[idx_v.at[idx_row]], bufs[b], gsems[b])

    def wait_gather(b):
        pltpu.make_async_copy(table_sp.at[idx_v.at[0]], bufs[b], gsems[b]).wait()

    pltpu.sync_copy(ts_hbm.at[pl.ds(row0, QK)], idx_v)
    for j in range(G):
        fire_gather(j, j)

    def step(t, carry):
        r = row0 + t * QK
        for j in range(QK):
            jb = j % NB
            fb = (j + G) % NB
            if j < QK - G:
                if j < G:
                    pl.when(t > 0)(lambda b=fb: drain_store(b))
                else:
                    drain_store(fb)
                fire_gather(fb, j + G)
            elif j == QK - G:
                drain_store(fb)
                nxt = jnp.minimum(t + 1, steps_per_w - 1)
                pltpu.sync_copy(ts_hbm.at[pl.ds(row0 + nxt * QK, QK)], idx_v)
                fire_gather(fb, 0)
            else:
                drain_store(fb)
                fire_gather(fb, 1)
            wait_gather(jb)
            pltpu.async_copy(bufs[jb], out_hbm.at[pl.ds((r + j) * IW, IW)],
                             ssems[jb])
        return carry

    lax.fori_loop(0, steps_per_w, step, 0)
    for j in range(G):
        wait_gather(j % NB)
    for j in range(QK - G, QK):
        drain_store(j % NB)


def kernel(timestamps, table):
    n, s = timestamps.shape
    b = n * s
    assert b % IW == 0
    idx2d = timestamps.reshape(b // IW, IW).astype(jnp.int32)

    info = plsc.get_sparse_core_info()
    nc, ns = info.num_cores, info.num_subcores
    nw = nc * ns
    assert (b // IW) % (nw * QK) == 0
    steps_per_w = (b // IW) // (nw * QK)

    mesh = plsc.VectorSubcoreMesh(core_axis_name="c", subcore_axis_name="s")
    out = pl.kernel(
        functools.partial(_gather_body, nc=nc, steps_per_w=steps_per_w),
        out_type=jax.ShapeDtypeStruct((b, EMB), jnp.float32),
        mesh=mesh,
        scratch_types=[pltpu.VMEM_SHARED(table.shape, jnp.float32),
                       pltpu.VMEM((QK, IW), jnp.int32)]
        + [pltpu.VMEM((IW, EMB), jnp.float32) for _ in range(NB)]
        + [pltpu.SemaphoreType.DMA for _ in range(2 * NB)],
    )(idx2d, table)
    return out.reshape(n, s, EMB)


# speedup vs baseline: 19.5895x; 1.0273x over previous
import functools

import jax
import jax.numpy as jnp
from jax import lax
from jax.experimental import pallas as pl
from jax.experimental.pallas import tpu as pltpu
from jax.experimental.pallas import tpu_sc as plsc

EMB = 128
IW = 128
HB = 16
QK = 2 * HB
NB = 4
G = 2


def _gather_body(ts_hbm, table_hbm, out_hbm, table_sp, idx_a, idx_b,
                 b0, b1, b2, b3, g0, g1, g2, g3, s0, s1, s2, s3,
                 sem_ia, sem_ib, *, nc, steps_per_w):
    sid = lax.axis_index("s")
    wid = sid * nc + lax.axis_index("c")
    row0 = wid * steps_per_w * QK
    bufs = (b0, b1, b2, b3)
    gsems = (g0, g1, g2, g3)
    ssems = (s0, s1, s2, s3)

    pl.when(sid == 0)(lambda: pltpu.sync_copy(table_hbm, table_sp))
    plsc.subcore_barrier()

    def drain_store(b):
        pltpu.make_async_copy(bufs[b], out_hbm.at[pl.ds(0, IW)], ssems[b]).wait()

    def idx_row(q):
        q = q % QK
        return idx_a.at[q % HB] if q < HB else idx_b.at[q - HB]

    def fire_gather(b, q):
        pltpu.async_copy(table_sp.at[idx_row(q)], bufs[b], gsems[b])

    def wait_gather(b):
        pltpu.make_async_copy(table_sp.at[idx_a.at[0]], bufs[b], gsems[b]).wait()

    def drain_idx(buf, sem):
        pltpu.make_async_copy(ts_hbm.at[pl.ds(0, HB)], buf, sem).wait()

    pltpu.sync_copy(ts_hbm.at[pl.ds(row0, HB)], idx_a)
    for j in range(G):
        fire_gather(j, j)

    def step(t, carry):
        r = row0 + t * QK
        for j in range(QK):
            jb = j % NB
            fb = (j + G) % NB
            if j == 0:
                pltpu.async_copy(ts_hbm.at[pl.ds(r + HB, HB)], idx_b, sem_ib)
            elif j == HB - G - 1:
                drain_idx(idx_b, sem_ib)
            elif j == HB:
                nxt = jnp.minimum(t + 1, steps_per_w - 1)
                pltpu.async_copy(ts_hbm.at[pl.ds(row0 + nxt * QK, HB)],
                                 idx_a, sem_ia)
            elif j == QK - G - 1:
                drain_idx(idx_a, sem_ia)
            if j < G:
                pl.when(t > 0)(lambda b=fb: drain_store(b))
            else:
                drain_store(fb)
            fire_gather(fb, j + G)
            wait_gather(jb)
            pltpu.async_copy(bufs[jb], out_hbm.at[pl.ds((r + j) * IW, IW)],
                             ssems[jb])
        return carry

    lax.fori_loop(0, steps_per_w, step, 0)
    for j in range(G):
        wait_gather(j % NB)
    for j in range(QK - G, QK):
        drain_store(j % NB)


def kernel(timestamps, table):
    n, s = timestamps.shape
    b = n * s
    assert b % IW == 0
    idx2d = timestamps.reshape(b // IW, IW).astype(jnp.int32)

    info = plsc.get_sparse_core_info()
    nc, ns = info.num_cores, info.num_subcores
    nw = nc * ns
    assert (b // IW) % (nw * QK) == 0
    steps_per_w = (b // IW) // (nw * QK)

    mesh = plsc.VectorSubcoreMesh(core_axis_name="c", subcore_axis_name="s")
    out = pl.kernel(
        functools.partial(_gather_body, nc=nc, steps_per_w=steps_per_w),
        out_type=jax.ShapeDtypeStruct((b, EMB), jnp.float32),
        mesh=mesh,
        scratch_types=[pltpu.VMEM_SHARED(table.shape, jnp.float32),
                       pltpu.VMEM((HB, IW), jnp.int32),
                       pltpu.VMEM((HB, IW), jnp.int32)]
        + [pltpu.VMEM((IW, EMB), jnp.float32) for _ in range(NB)]
        + [pltpu.SemaphoreType.DMA for _ in range(2 * NB + 2)],
    )(idx2d, table)
    return out.reshape(n, s, EMB)


# speedup vs baseline: 20.2482x; 1.0336x over previous
import functools

import jax
import jax.numpy as jnp
from jax import lax
from jax.experimental import pallas as pl
from jax.experimental.pallas import tpu as pltpu
from jax.experimental.pallas import tpu_sc as plsc

EMB = 128
SEQ = 200
C0, C1 = 128, 72
HB = 8
QK = 4 * HB
NB = 4
G = 2
CSZ = (C0, C1, C0, C1)


def _gather_body(ts_hbm, table_hbm, out_hbm, table_sp, idx_a, idx_b,
                 b0, b1, b2, b3, g0, g1, g2, g3, s0, s1, s2, s3,
                 sem_ia, sem_ib, *, nc, steps_per_w):
    sid = lax.axis_index("s")
    wid = sid * nc + lax.axis_index("c")
    row0 = wid * steps_per_w * (2 * HB)
    bufs = (b0, b1, b2, b3)
    gsems = (g0, g1, g2, g3)
    ssems = (s0, s1, s2, s3)

    pl.when(sid == 0)(lambda: pltpu.sync_copy(table_hbm, table_sp))
    plsc.subcore_barrier()

    def drain_store(b):
        pltpu.make_async_copy(bufs[b].at[pl.ds(0, CSZ[b])],
                              out_hbm.at[pl.ds(0, CSZ[b])], ssems[b]).wait()

    def idx_ref(q):
        q = q % QK
        buf, lrow = (idx_a, (q // 2) % HB) if (q // 2) % (2 * HB) < HB else \
                    (idx_b, (q // 2) % HB)
        if q % 2 == 0:
            return buf.at[lrow, pl.ds(0, C0)]
        return buf.at[lrow, pl.ds(C0, C1)]

    def fire_gather(b, q):
        pltpu.async_copy(table_sp.at[idx_ref(q)],
                         bufs[b].at[pl.ds(0, CSZ[b])], gsems[b])

    def wait_gather(b):
        pltpu.make_async_copy(table_sp.at[idx_a.at[0, pl.ds(0, CSZ[b])]],
                              bufs[b].at[pl.ds(0, CSZ[b])], gsems[b]).wait()

    def drain_idx(buf, sem):
        pltpu.make_async_copy(ts_hbm.at[pl.ds(0, HB)], buf, sem).wait()

    pltpu.sync_copy(ts_hbm.at[pl.ds(row0, HB)], idx_a)
    for j in range(G):
        fire_gather(j, j)

    def step(t, carry):
        r = row0 + t * 2 * HB
        for j in range(QK):
            jb = j % NB
            fb = (j + G) % NB
            if j == 0:
                pltpu.async_copy(ts_hbm.at[pl.ds(r + HB, HB)], idx_b, sem_ib)
            elif j == 2 * HB - G - 1:
                drain_idx(idx_b, sem_ib)
            elif j == 2 * HB:
                nxt = jnp.minimum(t + 1, steps_per_w - 1)
                pltpu.async_copy(ts_hbm.at[pl.ds(row0 + nxt * 2 * HB, HB)],
                                 idx_a, sem_ia)
            elif j == QK - G - 1:
                drain_idx(idx_a, sem_ia)
            if j < G:
                pl.when(t > 0)(lambda b=fb: drain_store(b))
            else:
                drain_store(fb)
            fire_gather(fb, j + G)
            wait_gather(jb)
            off = (r + j // 2) * SEQ + (j % 2) * C0
            pltpu.async_copy(bufs[jb].at[pl.ds(0, CSZ[jb])],
                             out_hbm.at[pl.ds(off, CSZ[jb])], ssems[jb])
        return carry

    lax.fori_loop(0, steps_per_w, step, 0)
    for j in range(G):
        wait_gather(j % NB)
    for j in range(QK - G, QK):
        drain_store(j % NB)


def kernel(timestamps, table):
    n, s = timestamps.shape
    assert s == SEQ
    b = n * s

    info = plsc.get_sparse_core_info()
    nc, ns = info.num_cores, info.num_subcores
    nw = nc * ns
    assert n % (nw * 2 * HB) == 0
    steps_per_w = n // (nw * 2 * HB)

    mesh = plsc.VectorSubcoreMesh(core_axis_name="c", subcore_axis_name="s")
    out = pl.kernel(
        functools.partial(_gather_body, nc=nc, steps_per_w=steps_per_w),
        out_type=jax.ShapeDtypeStruct((b, EMB), jnp.float32),
        mesh=mesh,
        scratch_types=[pltpu.VMEM_SHARED(table.shape, jnp.float32),
                       pltpu.VMEM((HB, SEQ), jnp.int32),
                       pltpu.VMEM((HB, SEQ), jnp.int32)]
        + [pltpu.VMEM((C0, EMB), jnp.float32) for _ in range(NB)]
        + [pltpu.SemaphoreType.DMA for _ in range(2 * NB + 2)],
    )(timestamps.astype(jnp.int32), table)
    return out.reshape(n, s, EMB)
